# baseline (speedup 1.0000x reference)
import functools

import jax
import jax.numpy as jnp
from jax import lax
from jax.experimental import pallas as pl
from jax.experimental.pallas import tpu as pltpu

_R = 128
_C = 256
_RA = 512
_GP = 64
_PADB = 127.0


def _mlp1_kernel(x_ref, w_ref, b_ref, o_ref):
    o_ref[...] = jnp.maximum(
        jnp.dot(x_ref[...], w_ref[...], preferred_element_type=jnp.float32)
        + b_ref[0:1, :], 0.0)


def _msg_kernel(lo_ref, hi_ref, rowmeta_ref, hrow_ref, colmeta_ref, h3_ref,
                w2_ref, b2_ref, wout_ref, bout_ref, o_ref,
                d2_ref, pool_ref, cnt_ref, *, radius2, hid):
    i = pl.program_id(0)
    lo = lo_ref[i]
    hi = hi_ref[i]

    rm = rowmeta_ref[...]
    rx, ry, rz, rb = rm[:, 0:1], rm[:, 1:2], rm[:, 2:3], rm[:, 3:4]
    rsq = rx * rx + ry * ry + rz * rz
    rid = i * _R + lax.broadcasted_iota(jnp.int32, (_R, _C), 0)

    def block_d2(b):
        cm = colmeta_ref[b]
        cx, cy, cz, cb = cm[0:1, :], cm[1:2, :], cm[2:3, :], cm[3:4, :]
        csq = cx * cx + cy * cy + cz * cz
        dotp = rx * cx + ry * cy + rz * cz
        d2 = jnp.maximum(rsq + csq - 2.0 * dotp, 0.0)
        cid = b * _C + lax.broadcasted_iota(jnp.int32, (_R, _C), 1)
        valid = (rb == cb) & (rid != cid)
        return d2, valid

    inf1 = jnp.full((_R, 1), jnp.inf, jnp.float32)

    def fill(b, m):
        d2, valid = block_d2(b)
        d2m = jnp.where(valid, d2, jnp.inf)
        d2_ref[b] = d2m
        return jnp.minimum(m, jnp.min(d2m, axis=1, keepdims=True))
    m0 = lax.fori_loop(lo, hi, fill, inf1)

    def select(k, carry):
        m, t8, t32, r = carry

        def killb(b, cm):
            c, m2 = cm
            blk = d2_ref[b]
            eq = blk == m
            blk2 = jnp.where(eq, jnp.inf, blk)
            d2_ref[b] = blk2
            c = c + jnp.sum(eq.astype(jnp.float32), axis=1, keepdims=True)
            return c, jnp.minimum(m2, jnp.min(blk2, axis=1, keepdims=True))
        c, m2 = lax.fori_loop(lo, hi, killb,
                              (jnp.zeros((_R, 1), jnp.float32), inf1))

        t8 = jnp.where((r < 8.0) & (r + c > 7.0), m, t8)
        t32 = jnp.where((r < 32.0) & (r + c > 31.0), m, t32)
        return m2, t8, t32, r + c

    _, t8, t32, _ = lax.fori_loop(
        0, 32, select, (m0, inf1, inf1, jnp.zeros((_R, 1), jnp.float32)))

    def aggb(b, carry):
        acc, cnt = carry
        d2, valid = block_d2(b)
        w = valid.astype(jnp.float32) * (
            (d2 <= t8).astype(jnp.float32)
            + ((d2 <= t32) & (d2 <= radius2)).astype(jnp.float32))
        acc = acc + jnp.dot(w, h3_ref[b], preferred_element_type=jnp.float32)
        return acc, cnt + jnp.sum(w, axis=1, keepdims=True)
    acc, cnt = lax.fori_loop(
        lo, hi, aggb,
        (jnp.zeros((_R, hid), jnp.float32), jnp.zeros((_R, 1), jnp.float32)))

    z = hrow_ref[...] + acc / jnp.maximum(cnt, 1.0)
    z = jnp.maximum(
        jnp.dot(z, w2_ref[...], preferred_element_type=jnp.float32)
        + b2_ref[0:1, :], 0.0)

    onehot = (rb == lax.broadcasted_iota(jnp.int32, (1, _GP), 1
                                         ).astype(jnp.float32)
              ).astype(jnp.float32)

    @pl.when(i == 0)
    def _():
        pool_ref[...] = jnp.zeros_like(pool_ref)
        cnt_ref[...] = jnp.zeros_like(cnt_ref)

    pool_ref[...] += lax.dot_general(
        onehot, z, (((0,), (0,)), ((), ())),
        preferred_element_type=jnp.float32)
    cnt_ref[...] += lax.dot_general(
        onehot, jnp.ones((_R, hid), jnp.float32), (((0,), (0,)), ((), ())),
        preferred_element_type=jnp.float32)

    @pl.when(i == pl.num_programs(0) - 1)
    def _():
        pooled = pool_ref[...] / jnp.maximum(cnt_ref[...], 1.0)
        o_ref[...] = (jnp.dot(pooled, wout_ref[...],
                              preferred_element_type=jnp.float32)
                      + bout_ref[0:1, :])


def _forward(x, pos, batch, W1, b1, W2, b2, Wout, bout, num_graphs, radius,
             interpret=False):
    n, in_f = x.shape
    hid = W1.shape[1]
    out_f = Wout.shape[1]
    npad = ((n + 1023) // 1024) * 1024
    nblk = npad // _C
    t = npad // _R

    xf = jnp.pad(x, ((0, npad - n), (0, 0)))
    posf = jnp.pad(pos, ((0, npad - n), (0, 0)))
    bf = jnp.pad(batch.astype(jnp.float32), (0, npad - n),
                 constant_values=_PADB)

    ra = min(_RA, npad)
    h = pl.pallas_call(
        _mlp1_kernel,
        grid=(npad // ra,),
        in_specs=[pl.BlockSpec((ra, in_f), lambda i: (i, 0)),
                  pl.BlockSpec((in_f, hid), lambda i: (0, 0)),
                  pl.BlockSpec((8, hid), lambda i: (0, 0))],
        out_specs=pl.BlockSpec((ra, hid), lambda i: (i, 0)),
        out_shape=jax.ShapeDtypeStruct((npad, hid), jnp.float32),
        interpret=interpret,
    )(xf, W1, jnp.broadcast_to(b1[None, :], (8, hid)))

    rowmeta = jnp.concatenate(
        [posf, bf[:, None], jnp.zeros((npad, 4), jnp.float32)], axis=1)
    colmeta = jnp.concatenate(
        [posf.T, bf[None, :], jnp.zeros((4, npad), jnp.float32)],
        axis=0).reshape(8, nblk, _C).transpose(1, 0, 2)
    h3 = h.reshape(nblk, _C, hid)

    bi = jnp.pad(batch.astype(jnp.int32), (0, npad - n),
                 constant_values=int(_PADB))
    g_lo = bi[::_R]
    g_hi = bi[_R - 1::_R]
    col_start = jnp.searchsorted(bi, g_lo, side='left').astype(jnp.int32)
    col_end = jnp.searchsorted(bi, g_hi, side='right').astype(jnp.int32)
    blk_lo = (col_start // _C).astype(jnp.int32)
    blk_hi = ((col_end + _C - 1) // _C).astype(jnp.int32)

    out = pl.pallas_call(
        functools.partial(_msg_kernel, radius2=float(radius) * float(radius),
                          hid=hid),
        grid=(t,),
        in_specs=[
            pl.BlockSpec(memory_space=pltpu.SMEM),
            pl.BlockSpec(memory_space=pltpu.SMEM),
            pl.BlockSpec((_R, 8), lambda i: (i, 0)),
            pl.BlockSpec((_R, hid), lambda i: (i, 0)),
            pl.BlockSpec((nblk, 8, _C), lambda i: (0, 0, 0)),
            pl.BlockSpec((nblk, _C, hid), lambda i: (0, 0, 0)),
            pl.BlockSpec((hid, hid), lambda i: (0, 0)),
            pl.BlockSpec((8, hid), lambda i: (0, 0)),
            pl.BlockSpec((hid, out_f), lambda i: (0, 0)),
            pl.BlockSpec((8, out_f), lambda i: (0, 0)),
        ],
        out_specs=pl.BlockSpec((_GP, out_f), lambda i: (0, 0)),
        out_shape=jax.ShapeDtypeStruct((_GP, out_f), jnp.float32),
        scratch_shapes=[
            pltpu.VMEM((nblk, _R, _C), jnp.float32),
            pltpu.VMEM((_GP, hid), jnp.float32),
            pltpu.VMEM((_GP, hid), jnp.float32),
        ],
        interpret=interpret,
    )(blk_lo, blk_hi, rowmeta, h, colmeta, h3, W2,
      jnp.broadcast_to(b2[None, :], (8, hid)), Wout,
      jnp.broadcast_to(bout[None, :], (8, out_f)))
    return out[:num_graphs]


@jax.jit
def kernel(x, pos, batch, W1, b1, W2, b2, Wout, bout):
    return _forward(x, pos, batch, W1, b1, W2, b2, Wout, bout,
                    num_graphs=50, radius=1.0)


# speedup vs baseline: 1.1811x; 1.1811x over previous
import functools

import jax
import jax.numpy as jnp
from jax import lax
from jax.experimental import pallas as pl
from jax.experimental.pallas import tpu as pltpu

_R = 128
_C = 512
_RA = 512
_GP = 64
_PADB = 127.0


def _mlp1_kernel(x_ref, w_ref, b_ref, o_ref):
    o_ref[...] = jnp.maximum(
        jnp.dot(x_ref[...], w_ref[...], preferred_element_type=jnp.float32)
        + b_ref[0:1, :], 0.0)


def _msg_kernel(lo_ref, hi_ref, rowmeta_ref, hrow_ref, colmeta_ref, h3_ref,
                w2_ref, b2_ref, wout_ref, bout_ref, o_ref,
                d2_ref, pool_ref, cnt_ref, *, radius2, hid):
    i = pl.program_id(0)
    lo = lo_ref[i]
    hi = hi_ref[i]

    rm = rowmeta_ref[...]
    rx, ry, rz, rb = rm[:, 0:1], rm[:, 1:2], rm[:, 2:3], rm[:, 3:4]
    rsq = rx * rx + ry * ry + rz * rz
    rid = i * _R + lax.broadcasted_iota(jnp.int32, (_R, _C), 0)

    def block_d2(b):
        cm = colmeta_ref[b]
        cx, cy, cz, cb = cm[0:1, :], cm[1:2, :], cm[2:3, :], cm[3:4, :]
        csq = cx * cx + cy * cy + cz * cz
        dotp = rx * cx + ry * cy + rz * cz
        d2 = jnp.maximum(rsq + csq - 2.0 * dotp, 0.0)
        cid = b * _C + lax.broadcasted_iota(jnp.int32, (_R, _C), 1)
        valid = (rb == cb) & (rid != cid)
        return d2, valid

    inf1 = jnp.full((_R, 1), jnp.inf, jnp.float32)

    def fill(b, m):
        d2, valid = block_d2(b)
        d2m = jnp.where(valid, d2, jnp.inf)
        d2_ref[b] = d2m
        return jnp.minimum(m, jnp.min(d2m, axis=1, keepdims=True))
    m0 = lax.fori_loop(lo, hi, fill, inf1)

    def select(k, carry):
        m, t8, t32, r = carry

        def killb(b, cm):
            c, m2 = cm
            blk = d2_ref[b]
            eq = blk == m
            blk2 = jnp.where(eq, jnp.inf, blk)
            d2_ref[b] = blk2
            c = c + jnp.sum(eq.astype(jnp.float32), axis=1, keepdims=True)
            return c, jnp.minimum(m2, jnp.min(blk2, axis=1, keepdims=True))
        c, m2 = lax.fori_loop(lo, hi, killb,
                              (jnp.zeros((_R, 1), jnp.float32), inf1))

        t8 = jnp.where((r < 8.0) & (r + c > 7.0), m, t8)
        t32 = jnp.where((r < 32.0) & (r + c > 31.0), m, t32)
        return m2, t8, t32, r + c

    _, t8, t32, _ = lax.fori_loop(
        0, 32, select, (m0, inf1, inf1, jnp.zeros((_R, 1), jnp.float32)))

    def aggb(b, carry):
        acc, cnt = carry
        d2, valid = block_d2(b)
        w = valid.astype(jnp.float32) * (
            (d2 <= t8).astype(jnp.float32)
            + ((d2 <= t32) & (d2 <= radius2)).astype(jnp.float32))
        acc = acc + jnp.dot(w, h3_ref[b], preferred_element_type=jnp.float32)
        return acc, cnt + jnp.sum(w, axis=1, keepdims=True)
    acc, cnt = lax.fori_loop(
        lo, hi, aggb,
        (jnp.zeros((_R, hid), jnp.float32), jnp.zeros((_R, 1), jnp.float32)))

    z = hrow_ref[...] + acc / jnp.maximum(cnt, 1.0)
    z = jnp.maximum(
        jnp.dot(z, w2_ref[...], preferred_element_type=jnp.float32)
        + b2_ref[0:1, :], 0.0)

    onehot = (rb == lax.broadcasted_iota(jnp.int32, (1, _GP), 1
                                         ).astype(jnp.float32)
              ).astype(jnp.float32)

    @pl.when(i == 0)
    def _():
        pool_ref[...] = jnp.zeros_like(pool_ref)
        cnt_ref[...] = jnp.zeros_like(cnt_ref)

    pool_ref[...] += lax.dot_general(
        onehot, z, (((0,), (0,)), ((), ())),
        preferred_element_type=jnp.float32)
    cnt_ref[...] += lax.dot_general(
        onehot, jnp.ones((_R, hid), jnp.float32), (((0,), (0,)), ((), ())),
        preferred_element_type=jnp.float32)

    @pl.when(i == pl.num_programs(0) - 1)
    def _():
        pooled = pool_ref[...] / jnp.maximum(cnt_ref[...], 1.0)
        o_ref[...] = (jnp.dot(pooled, wout_ref[...],
                              preferred_element_type=jnp.float32)
                      + bout_ref[0:1, :])


def _forward(x, pos, batch, W1, b1, W2, b2, Wout, bout, num_graphs, radius,
             interpret=False):
    n, in_f = x.shape
    hid = W1.shape[1]
    out_f = Wout.shape[1]
    npad = ((n + 1023) // 1024) * 1024
    nblk = npad // _C
    t = npad // _R

    xf = jnp.pad(x, ((0, npad - n), (0, 0)))
    posf = jnp.pad(pos, ((0, npad - n), (0, 0)))
    bf = jnp.pad(batch.astype(jnp.float32), (0, npad - n),
                 constant_values=_PADB)

    ra = min(_RA, npad)
    h = pl.pallas_call(
        _mlp1_kernel,
        grid=(npad // ra,),
        in_specs=[pl.BlockSpec((ra, in_f), lambda i: (i, 0)),
                  pl.BlockSpec((in_f, hid), lambda i: (0, 0)),
                  pl.BlockSpec((8, hid), lambda i: (0, 0))],
        out_specs=pl.BlockSpec((ra, hid), lambda i: (i, 0)),
        out_shape=jax.ShapeDtypeStruct((npad, hid), jnp.float32),
        interpret=interpret,
    )(xf, W1, jnp.broadcast_to(b1[None, :], (8, hid)))

    rowmeta = jnp.concatenate(
        [posf, bf[:, None], jnp.zeros((npad, 4), jnp.float32)], axis=1)
    colmeta = jnp.concatenate(
        [posf.T, bf[None, :], jnp.zeros((4, npad), jnp.float32)],
        axis=0).reshape(8, nblk, _C).transpose(1, 0, 2)
    h3 = h.reshape(nblk, _C, hid)

    bi = jnp.pad(batch.astype(jnp.int32), (0, npad - n),
                 constant_values=int(_PADB))
    g_lo = bi[::_R]
    g_hi = bi[_R - 1::_R]
    col_start = jnp.searchsorted(bi, g_lo, side='left').astype(jnp.int32)
    col_end = jnp.searchsorted(bi, g_hi, side='right').astype(jnp.int32)
    blk_lo = (col_start // _C).astype(jnp.int32)
    blk_hi = ((col_end + _C - 1) // _C).astype(jnp.int32)

    out = pl.pallas_call(
        functools.partial(_msg_kernel, radius2=float(radius) * float(radius),
                          hid=hid),
        grid=(t,),
        in_specs=[
            pl.BlockSpec(memory_space=pltpu.SMEM),
            pl.BlockSpec(memory_space=pltpu.SMEM),
            pl.BlockSpec((_R, 8), lambda i: (i, 0)),
            pl.BlockSpec((_R, hid), lambda i: (i, 0)),
            pl.BlockSpec((nblk, 8, _C), lambda i: (0, 0, 0)),
            pl.BlockSpec((nblk, _C, hid), lambda i: (0, 0, 0)),
            pl.BlockSpec((hid, hid), lambda i: (0, 0)),
            pl.BlockSpec((8, hid), lambda i: (0, 0)),
            pl.BlockSpec((hid, out_f), lambda i: (0, 0)),
            pl.BlockSpec((8, out_f), lambda i: (0, 0)),
        ],
        out_specs=pl.BlockSpec((_GP, out_f), lambda i: (0, 0)),
        out_shape=jax.ShapeDtypeStruct((_GP, out_f), jnp.float32),
        scratch_shapes=[
            pltpu.VMEM((nblk, _R, _C), jnp.float32),
            pltpu.VMEM((_GP, hid), jnp.float32),
            pltpu.VMEM((_GP, hid), jnp.float32),
        ],
        interpret=interpret,
    )(blk_lo, blk_hi, rowmeta, h, colmeta, h3, W2,
      jnp.broadcast_to(b2[None, :], (8, hid)), Wout,
      jnp.broadcast_to(bout[None, :], (8, out_f)))
    return out[:num_graphs]


@jax.jit
def kernel(x, pos, batch, W1, b1, W2, b2, Wout, bout):
    return _forward(x, pos, batch, W1, b1, W2, b2, Wout, bout,
                    num_graphs=50, radius=1.0)


# speedup vs baseline: 1.2534x; 1.0612x over previous
import functools

import jax
import jax.numpy as jnp
from jax import lax
from jax.experimental import pallas as pl
from jax.experimental.pallas import tpu as pltpu

_R = 128
_C = 512
_RA = 512
_GP = 64
_PADB = 127.0


def _mlp1_kernel(x_ref, w_ref, b_ref, o_ref):
    o_ref[...] = jnp.maximum(
        jnp.dot(x_ref[...], w_ref[...], preferred_element_type=jnp.float32)
        + b_ref[0:1, :], 0.0)


def _msg_kernel(lo_ref, hi_ref, rowmeta_ref, hrow_ref, colmeta_ref, h3_ref,
                w2_ref, b2_ref, wout_ref, bout_ref, o_ref,
                d2_ref, pool_ref, cnt_ref, *, radius2, hid):
    i = pl.program_id(0)
    lo = lo_ref[i]
    hi = hi_ref[i]

    rm = rowmeta_ref[...]
    rx, ry, rz, rb = rm[:, 0:1], rm[:, 1:2], rm[:, 2:3], rm[:, 3:4]
    rsq = rx * rx + ry * ry + rz * rz
    rid = i * _R + lax.broadcasted_iota(jnp.int32, (_R, _C), 0)

    def block_d2(b):
        cm = colmeta_ref[b]
        cx, cy, cz, cb = cm[0:1, :], cm[1:2, :], cm[2:3, :], cm[3:4, :]
        csq = cx * cx + cy * cy + cz * cz
        dotp = rx * cx + ry * cy + rz * cz
        d2 = jnp.maximum(rsq + csq - 2.0 * dotp, 0.0)
        cid = b * _C + lax.broadcasted_iota(jnp.int32, (_R, _C), 1)
        valid = (rb == cb) & (rid != cid)
        return d2, valid

    inf1 = jnp.full((_R, 1), jnp.inf, jnp.float32)

    def two_smallest(blk):
        v1 = jnp.min(blk, axis=1, keepdims=True)
        v2 = jnp.min(jnp.where(blk == v1, jnp.inf, blk), axis=1, keepdims=True)
        return v1, v2

    def merge2(m1c, m2c, v1, v2):
        m1n = jnp.minimum(m1c, v1)

        def gt(x):
            return jnp.where(x > m1n, x, jnp.inf)
        m2n = jnp.minimum(jnp.minimum(gt(m1c), gt(m2c)),
                          jnp.minimum(gt(v1), gt(v2)))
        return m1n, m2n

    def fill(b, cm):
        d2, valid = block_d2(b)
        d2m = jnp.where(valid, d2, jnp.inf)
        d2_ref[b] = d2m
        v1, v2 = two_smallest(d2m)
        return merge2(cm[0], cm[1], v1, v2)
    m1, m2 = lax.fori_loop(lo, hi, fill, (inf1, inf1))

    def select(k, carry):
        m1, m2, t8, t32, r = carry

        def killb(b, cm):
            c1, c2, n1, n2 = cm
            blk = d2_ref[b]
            eq1 = blk == m1
            eq2 = blk == m2
            blk2 = jnp.where(eq1 | eq2, jnp.inf, blk)
            d2_ref[b] = blk2
            c1 = c1 + jnp.sum(eq1.astype(jnp.float32), axis=1, keepdims=True)
            c2 = c2 + jnp.sum(eq2.astype(jnp.float32), axis=1, keepdims=True)
            v1, v2 = two_smallest(blk2)
            n1, n2 = merge2(n1, n2, v1, v2)
            return c1, c2, n1, n2
        z1 = jnp.zeros((_R, 1), jnp.float32)
        c1, c2, n1, n2 = lax.fori_loop(lo, hi, killb, (z1, z1, inf1, inf1))

        r1 = r + c1
        t8 = jnp.where((r < 8.0) & (r1 > 7.0), m1, t8)
        t8 = jnp.where((r1 < 8.0) & (r1 + c2 > 7.0), m2, t8)
        t32 = jnp.where((r < 32.0) & (r1 > 31.0), m1, t32)
        t32 = jnp.where((r1 < 32.0) & (r1 + c2 > 31.0), m2, t32)
        return n1, n2, t8, t32, r1 + c2

    _, _, t8, t32, _ = lax.fori_loop(
        0, 16, select,
        (m1, m2, inf1, inf1, jnp.zeros((_R, 1), jnp.float32)))

    def aggb(b, carry):
        acc, cnt = carry
        d2, valid = block_d2(b)
        w = valid.astype(jnp.float32) * (
            (d2 <= t8).astype(jnp.float32)
            + ((d2 <= t32) & (d2 <= radius2)).astype(jnp.float32))
        acc = acc + jnp.dot(w, h3_ref[b], preferred_element_type=jnp.float32)
        return acc, cnt + jnp.sum(w, axis=1, keepdims=True)
    acc, cnt = lax.fori_loop(
        lo, hi, aggb,
        (jnp.zeros((_R, hid), jnp.float32), jnp.zeros((_R, 1), jnp.float32)))

    z = hrow_ref[...] + acc / jnp.maximum(cnt, 1.0)
    z = jnp.maximum(
        jnp.dot(z, w2_ref[...], preferred_element_type=jnp.float32)
        + b2_ref[0:1, :], 0.0)

    onehot = (rb == lax.broadcasted_iota(jnp.int32, (1, _GP), 1
                                         ).astype(jnp.float32)
              ).astype(jnp.float32)

    @pl.when(i == 0)
    def _():
        pool_ref[...] = jnp.zeros_like(pool_ref)
        cnt_ref[...] = jnp.zeros_like(cnt_ref)

    pool_ref[...] += lax.dot_general(
        onehot, z, (((0,), (0,)), ((), ())),
        preferred_element_type=jnp.float32)
    cnt_ref[...] += lax.dot_general(
        onehot, jnp.ones((_R, hid), jnp.float32), (((0,), (0,)), ((), ())),
        preferred_element_type=jnp.float32)

    @pl.when(i == pl.num_programs(0) - 1)
    def _():
        pooled = pool_ref[...] / jnp.maximum(cnt_ref[...], 1.0)
        o_ref[...] = (jnp.dot(pooled, wout_ref[...],
                              preferred_element_type=jnp.float32)
                      + bout_ref[0:1, :])


def _forward(x, pos, batch, W1, b1, W2, b2, Wout, bout, num_graphs, radius,
             interpret=False):
    n, in_f = x.shape
    hid = W1.shape[1]
    out_f = Wout.shape[1]
    npad = ((n + 1023) // 1024) * 1024
    nblk = npad // _C
    t = npad // _R

    xf = jnp.pad(x, ((0, npad - n), (0, 0)))
    posf = jnp.pad(pos, ((0, npad - n), (0, 0)))
    bf = jnp.pad(batch.astype(jnp.float32), (0, npad - n),
                 constant_values=_PADB)

    ra = min(_RA, npad)
    h = pl.pallas_call(
        _mlp1_kernel,
        grid=(npad // ra,),
        in_specs=[pl.BlockSpec((ra, in_f), lambda i: (i, 0)),
                  pl.BlockSpec((in_f, hid), lambda i: (0, 0)),
                  pl.BlockSpec((8, hid), lambda i: (0, 0))],
        out_specs=pl.BlockSpec((ra, hid), lambda i: (i, 0)),
        out_shape=jax.ShapeDtypeStruct((npad, hid), jnp.float32),
        interpret=interpret,
    )(xf, W1, jnp.broadcast_to(b1[None, :], (8, hid)))

    rowmeta = jnp.concatenate(
        [posf, bf[:, None], jnp.zeros((npad, 4), jnp.float32)], axis=1)
    colmeta = jnp.concatenate(
        [posf.T, bf[None, :], jnp.zeros((4, npad), jnp.float32)],
        axis=0).reshape(8, nblk, _C).transpose(1, 0, 2)
    h3 = h.reshape(nblk, _C, hid)

    bi = jnp.pad(batch.astype(jnp.int32), (0, npad - n),
                 constant_values=int(_PADB))
    g_lo = bi[::_R]
    g_hi = bi[_R - 1::_R]
    col_start = jnp.searchsorted(bi, g_lo, side='left').astype(jnp.int32)
    col_end = jnp.searchsorted(bi, g_hi, side='right').astype(jnp.int32)
    blk_lo = (col_start // _C).astype(jnp.int32)
    blk_hi = ((col_end + _C - 1) // _C).astype(jnp.int32)

    out = pl.pallas_call(
        functools.partial(_msg_kernel, radius2=float(radius) * float(radius),
                          hid=hid),
        grid=(t,),
        in_specs=[
            pl.BlockSpec(memory_space=pltpu.SMEM),
            pl.BlockSpec(memory_space=pltpu.SMEM),
            pl.BlockSpec((_R, 8), lambda i: (i, 0)),
            pl.BlockSpec((_R, hid), lambda i: (i, 0)),
            pl.BlockSpec((nblk, 8, _C), lambda i: (0, 0, 0)),
            pl.BlockSpec((nblk, _C, hid), lambda i: (0, 0, 0)),
            pl.BlockSpec((hid, hid), lambda i: (0, 0)),
            pl.BlockSpec((8, hid), lambda i: (0, 0)),
            pl.BlockSpec((hid, out_f), lambda i: (0, 0)),
            pl.BlockSpec((8, out_f), lambda i: (0, 0)),
        ],
        out_specs=pl.BlockSpec((_GP, out_f), lambda i: (0, 0)),
        out_shape=jax.ShapeDtypeStruct((_GP, out_f), jnp.float32),
        scratch_shapes=[
            pltpu.VMEM((nblk, _R, _C), jnp.float32),
            pltpu.VMEM((_GP, hid), jnp.float32),
            pltpu.VMEM((_GP, hid), jnp.float32),
        ],
        interpret=interpret,
    )(blk_lo, blk_hi, rowmeta, h, colmeta, h3, W2,
      jnp.broadcast_to(b2[None, :], (8, hid)), Wout,
      jnp.broadcast_to(bout[None, :], (8, out_f)))
    return out[:num_graphs]


@jax.jit
def kernel(x, pos, batch, W1, b1, W2, b2, Wout, bout):
    return _forward(x, pos, batch, W1, b1, W2, b2, Wout, bout,
                    num_graphs=50, radius=1.0)


# speedup vs baseline: 1.2944x; 1.0327x over previous
import functools

import jax
import jax.numpy as jnp
from jax import lax
from jax.experimental import pallas as pl
from jax.experimental.pallas import tpu as pltpu

_R = 128
_C = 512
_RA = 512
_GP = 64
_PADB = 127.0


def _mlp1_kernel(x_ref, w_ref, b_ref, o_ref):
    o_ref[...] = jnp.maximum(
        jnp.dot(x_ref[...], w_ref[...], preferred_element_type=jnp.float32)
        + b_ref[0:1, :], 0.0)


def _msg_kernel(lo_ref, hi_ref, rowmeta_ref, hrow_ref, colmeta_ref, h3_ref,
                w2_ref, b2_ref, wout_ref, bout_ref, o_ref,
                d2_ref, pool_ref, cnt_ref, *, radius2, hid):
    i = pl.program_id(0)
    lo = lo_ref[i]
    hi = hi_ref[i]

    rm = rowmeta_ref[...]
    rx, ry, rz, rb = rm[:, 0:1], rm[:, 1:2], rm[:, 2:3], rm[:, 3:4]
    rsq = rx * rx + ry * ry + rz * rz
    rid = i * _R + lax.broadcasted_iota(jnp.int32, (_R, _C), 0)

    def block_d2(b):
        cm = colmeta_ref[b]
        cx, cy, cz, cb = cm[0:1, :], cm[1:2, :], cm[2:3, :], cm[3:4, :]
        csq = cx * cx + cy * cy + cz * cz
        dotp = rx * cx + ry * cy + rz * cz
        d2 = jnp.maximum(rsq + csq - 2.0 * dotp, 0.0)
        cid = b * _C + lax.broadcasted_iota(jnp.int32, (_R, _C), 1)
        valid = (rb == cb) & (rid != cid)
        return d2, valid

    inf1 = jnp.full((_R, 1), jnp.inf, jnp.float32)

    def two_smallest(blk):
        v1 = jnp.min(blk, axis=1, keepdims=True)
        v2 = jnp.min(jnp.where(blk == v1, jnp.inf, blk), axis=1, keepdims=True)
        return v1, v2

    def merge2(m1c, m2c, v1, v2):
        m1n = jnp.minimum(m1c, v1)

        def gt(x):
            return jnp.where(x > m1n, x, jnp.inf)
        m2n = jnp.minimum(jnp.minimum(gt(m1c), gt(m2c)),
                          jnp.minimum(gt(v1), gt(v2)))
        return m1n, m2n

    def fill(b, cm):
        d2, valid = block_d2(b)
        d2m = jnp.where(valid, d2, jnp.inf)
        d2_ref[b] = d2m
        v1, v2 = two_smallest(d2m)
        return merge2(cm[0], cm[1], v1, v2)
    m1, m2 = lax.fori_loop(lo, hi, fill, (inf1, inf1))

    def select(k, carry):
        m1, m2, t8, t32, r = carry

        ones8 = jnp.ones((_C, 8), jnp.float32)

        def killb(b, cm):
            cc, n1, n2 = cm
            blk = d2_ref[b]
            le2 = blk <= m2
            blk2 = jnp.where(le2, jnp.inf, blk)
            d2_ref[b] = blk2
            w = (blk <= m1).astype(jnp.float32) + 1024.0 * le2.astype(
                jnp.float32)
            cc = cc + jnp.dot(w, ones8,
                              preferred_element_type=jnp.float32)[:, 0:1]
            v1, v2 = two_smallest(blk2)
            n1, n2 = merge2(n1, n2, v1, v2)
            return cc, n1, n2
        z1 = jnp.zeros((_R, 1), jnp.float32)
        cc, n1, n2 = lax.fori_loop(lo, hi, killb, (z1, inf1, inf1))
        tot2 = jnp.floor(cc * (1.0 / 1024.0))
        c1 = cc - 1024.0 * tot2
        c2 = tot2 - c1

        r1 = r + c1
        t8 = jnp.where((r < 8.0) & (r1 > 7.0), m1, t8)
        t8 = jnp.where((r1 < 8.0) & (r1 + c2 > 7.0), m2, t8)
        t32 = jnp.where((r < 32.0) & (r1 > 31.0), m1, t32)
        t32 = jnp.where((r1 < 32.0) & (r1 + c2 > 31.0), m2, t32)
        return n1, n2, t8, t32, r1 + c2

    _, _, t8, t32, _ = lax.fori_loop(
        0, 16, select,
        (m1, m2, inf1, inf1, jnp.zeros((_R, 1), jnp.float32)))

    def aggb(b, carry):
        acc, cnt = carry
        d2, valid = block_d2(b)
        w = valid.astype(jnp.float32) * (
            (d2 <= t8).astype(jnp.float32)
            + ((d2 <= t32) & (d2 <= radius2)).astype(jnp.float32))
        acc = acc + jnp.dot(w, h3_ref[b], preferred_element_type=jnp.float32)
        return acc, cnt + jnp.sum(w, axis=1, keepdims=True)
    acc, cnt = lax.fori_loop(
        lo, hi, aggb,
        (jnp.zeros((_R, hid), jnp.float32), jnp.zeros((_R, 1), jnp.float32)))

    z = hrow_ref[...] + acc / jnp.maximum(cnt, 1.0)
    z = jnp.maximum(
        jnp.dot(z, w2_ref[...], preferred_element_type=jnp.float32)
        + b2_ref[0:1, :], 0.0)

    onehot = (rb == lax.broadcasted_iota(jnp.int32, (1, _GP), 1
                                         ).astype(jnp.float32)
              ).astype(jnp.float32)

    @pl.when(i == 0)
    def _():
        pool_ref[...] = jnp.zeros_like(pool_ref)
        cnt_ref[...] = jnp.zeros_like(cnt_ref)

    pool_ref[...] += lax.dot_general(
        onehot, z, (((0,), (0,)), ((), ())),
        preferred_element_type=jnp.float32)
    cnt_ref[...] += lax.dot_general(
        onehot, jnp.ones((_R, hid), jnp.float32), (((0,), (0,)), ((), ())),
        preferred_element_type=jnp.float32)

    @pl.when(i == pl.num_programs(0) - 1)
    def _():
        pooled = pool_ref[...] / jnp.maximum(cnt_ref[...], 1.0)
        o_ref[...] = (jnp.dot(pooled, wout_ref[...],
                              preferred_element_type=jnp.float32)
                      + bout_ref[0:1, :])


def _forward(x, pos, batch, W1, b1, W2, b2, Wout, bout, num_graphs, radius,
             interpret=False):
    n, in_f = x.shape
    hid = W1.shape[1]
    out_f = Wout.shape[1]
    npad = ((n + 1023) // 1024) * 1024
    nblk = npad // _C
    t = npad // _R

    xf = jnp.pad(x, ((0, npad - n), (0, 0)))
    posf = jnp.pad(pos, ((0, npad - n), (0, 0)))
    bf = jnp.pad(batch.astype(jnp.float32), (0, npad - n),
                 constant_values=_PADB)

    ra = min(_RA, npad)
    h = pl.pallas_call(
        _mlp1_kernel,
        grid=(npad // ra,),
        in_specs=[pl.BlockSpec((ra, in_f), lambda i: (i, 0)),
                  pl.BlockSpec((in_f, hid), lambda i: (0, 0)),
                  pl.BlockSpec((8, hid), lambda i: (0, 0))],
        out_specs=pl.BlockSpec((ra, hid), lambda i: (i, 0)),
        out_shape=jax.ShapeDtypeStruct((npad, hid), jnp.float32),
        interpret=interpret,
    )(xf, W1, jnp.broadcast_to(b1[None, :], (8, hid)))

    rowmeta = jnp.concatenate(
        [posf, bf[:, None], jnp.zeros((npad, 4), jnp.float32)], axis=1)
    colmeta = jnp.concatenate(
        [posf.T, bf[None, :], jnp.zeros((4, npad), jnp.float32)],
        axis=0).reshape(8, nblk, _C).transpose(1, 0, 2)
    h3 = h.reshape(nblk, _C, hid)

    bi = jnp.pad(batch.astype(jnp.int32), (0, npad - n),
                 constant_values=int(_PADB))
    g_lo = bi[::_R]
    g_hi = bi[_R - 1::_R]
    col_start = jnp.searchsorted(bi, g_lo, side='left').astype(jnp.int32)
    col_end = jnp.searchsorted(bi, g_hi, side='right').astype(jnp.int32)
    blk_lo = (col_start // _C).astype(jnp.int32)
    blk_hi = ((col_end + _C - 1) // _C).astype(jnp.int32)

    out = pl.pallas_call(
        functools.partial(_msg_kernel, radius2=float(radius) * float(radius),
                          hid=hid),
        grid=(t,),
        in_specs=[
            pl.BlockSpec(memory_space=pltpu.SMEM),
            pl.BlockSpec(memory_space=pltpu.SMEM),
            pl.BlockSpec((_R, 8), lambda i: (i, 0)),
            pl.BlockSpec((_R, hid), lambda i: (i, 0)),
            pl.BlockSpec((nblk, 8, _C), lambda i: (0, 0, 0)),
            pl.BlockSpec((nblk, _C, hid), lambda i: (0, 0, 0)),
            pl.BlockSpec((hid, hid), lambda i: (0, 0)),
            pl.BlockSpec((8, hid), lambda i: (0, 0)),
            pl.BlockSpec((hid, out_f), lambda i: (0, 0)),
            pl.BlockSpec((8, out_f), lambda i: (0, 0)),
        ],
        out_specs=pl.BlockSpec((_GP, out_f), lambda i: (0, 0)),
        out_shape=jax.ShapeDtypeStruct((_GP, out_f), jnp.float32),
        scratch_shapes=[
            pltpu.VMEM((nblk, _R, _C), jnp.float32),
            pltpu.VMEM((_GP, hid), jnp.float32),
            pltpu.VMEM((_GP, hid), jnp.float32),
        ],
        interpret=interpret,
    )(blk_lo, blk_hi, rowmeta, h, colmeta, h3, W2,
      jnp.broadcast_to(b2[None, :], (8, hid)), Wout,
      jnp.broadcast_to(bout[None, :], (8, out_f)))
    return out[:num_graphs]


@jax.jit
def kernel(x, pos, batch, W1, b1, W2, b2, Wout, bout):
    return _forward(x, pos, batch, W1, b1, W2, b2, Wout, bout,
                    num_graphs=50, radius=1.0)
